# baseline (device time: 13549 ns/iter reference)
import jax
import jax.numpy as jnp
from jax import lax
from jax.experimental import pallas as pl
from jax.experimental.pallas import tpu as pltpu

Z = 4


def kernel(x, W, labels):
    t, d = x.shape
    v_local = W.shape[1]

    def body(x_ref, w_ref, labels_ref, out_ref, gbuf, send_sems, recv_sems):
        my_x = lax.axis_index("x")
        my_y = lax.axis_index("y")
        my_z = lax.axis_index("z")

        barrier_sem = pltpu.get_barrier_semaphore()
        for dz in range(1, Z):
            peer = lax.rem(my_z + dz, Z)
            pl.semaphore_signal(
                barrier_sem,
                inc=1,
                device_id=(my_x, my_y, peer),
                device_id_type=pl.DeviceIdType.MESH,
            )

        logits = jnp.dot(
            x_ref[:, :], w_ref[:, :], preferred_element_type=jnp.float32
        )
        s = jnp.sum(jnp.exp(logits), axis=1)
        col = lax.broadcasted_iota(jnp.int32, (t, v_local), 1) + my_z * v_local
        lab = jnp.sum(
            jnp.where(col == labels_ref[:].reshape(t, 1), logits, 0.0), axis=1
        )
        gbuf[my_z, 0, :] = s
        gbuf[my_z, 1, :] = lab

        pl.semaphore_wait(barrier_sem, Z - 1)

        sends = []
        for dz in range(Z - 1, 0, -1):
            peer = lax.rem(my_z + dz, Z)
            rdma = pltpu.make_async_remote_copy(
                src_ref=gbuf.at[my_z],
                dst_ref=gbuf.at[my_z],
                send_sem=send_sems.at[dz - 1],
                recv_sem=recv_sems.at[dz - 1],
                device_id=(my_x, my_y, peer),
                device_id_type=pl.DeviceIdType.MESH,
            )
            rdma.start()
            sends.append(rdma)

        def wait_inbound(dz):
            origin = lax.rem(my_z - dz + Z, Z)
            recv = pltpu.make_async_remote_copy(
                src_ref=gbuf.at[origin],
                dst_ref=gbuf.at[origin],
                send_sem=send_sems.at[dz - 1],
                recv_sem=recv_sems.at[dz - 1],
                device_id=(my_x, my_y, my_z),
                device_id_type=pl.DeviceIdType.MESH,
            )
            recv.wait_recv()
            return origin

        o1 = wait_inbound(1)
        o2 = wait_inbound(2)
        s_near = s + gbuf[o1, 0, :] + gbuf[o2, 0, :]
        lab_near = lab + gbuf[o1, 1, :] + gbuf[o2, 1, :]
        o3 = wait_inbound(3)
        out_ref[:] = jnp.log(s_near + gbuf[o3, 0, :]) - (
            lab_near + gbuf[o3, 1, :]
        )

        for rdma in sends:
            rdma.wait_send()

    return pl.pallas_call(
        body,
        out_shape=jax.ShapeDtypeStruct((t,), jnp.float32),
        in_specs=[
            pl.BlockSpec(memory_space=pltpu.VMEM),
            pl.BlockSpec(memory_space=pltpu.VMEM),
            pl.BlockSpec(memory_space=pltpu.VMEM),
        ],
        out_specs=pl.BlockSpec(memory_space=pltpu.VMEM),
        scratch_shapes=[
            pltpu.VMEM((Z, 2, t), jnp.float32),
            pltpu.SemaphoreType.DMA((Z - 1,)),
            pltpu.SemaphoreType.DMA((Z - 1,)),
        ],
        compiler_params=pltpu.CompilerParams(collective_id=0),
    )(x, W, labels)


# device time: 13079 ns/iter; 1.0359x vs baseline; 1.0359x over previous
import jax
import jax.numpy as jnp
from jax import lax
from jax.experimental import pallas as pl
from jax.experimental.pallas import tpu as pltpu

Z = 4


def kernel(x, W, labels):
    t, d = x.shape
    v_local = W.shape[1]

    def body(x_ref, w_ref, labels_ref, out_ref, gbuf, send_sems, recv_sems):
        my_x = lax.axis_index("x")
        my_y = lax.axis_index("y")
        my_z = lax.axis_index("z")

        barrier_sem = pltpu.get_barrier_semaphore()
        for dz in range(1, Z):
            peer = lax.rem(my_z + dz, Z)
            pl.semaphore_signal(
                barrier_sem,
                inc=1,
                device_id=(my_x, my_y, peer),
                device_id_type=pl.DeviceIdType.MESH,
            )

        logits = jnp.dot(
            x_ref[:, :], w_ref[:, :], preferred_element_type=jnp.float32
        )
        s = jnp.sum(jnp.exp(logits), axis=1)
        col = lax.broadcasted_iota(jnp.int32, (t, v_local), 1) + my_z * v_local
        lab = jnp.sum(
            jnp.where(col == labels_ref[:].reshape(t, 1), logits, 0.0), axis=1
        )
        gbuf[my_z, 0, :] = s
        gbuf[my_z, 1, :] = lab

        pl.semaphore_wait(barrier_sem, Z - 1)

        sends = []
        for dz in range(Z - 1, 0, -1):
            peer = lax.rem(my_z + dz, Z)
            rdma = pltpu.make_async_remote_copy(
                src_ref=gbuf.at[my_z],
                dst_ref=gbuf.at[my_z],
                send_sem=send_sems.at[dz - 1],
                recv_sem=recv_sems.at[dz - 1],
                device_id=(my_x, my_y, peer),
                device_id_type=pl.DeviceIdType.MESH,
            )
            rdma.start()
            sends.append(rdma)

        for dz in range(1, Z):
            origin = lax.rem(my_z - dz + Z, Z)
            recv = pltpu.make_async_remote_copy(
                src_ref=gbuf.at[origin],
                dst_ref=gbuf.at[origin],
                send_sem=send_sems.at[dz - 1],
                recv_sem=recv_sems.at[dz - 1],
                device_id=(my_x, my_y, my_z),
                device_id_type=pl.DeviceIdType.MESH,
            )
            recv.wait_recv()

        out_ref[:] = jnp.log(jnp.sum(gbuf[:, 0, :], axis=0)) - jnp.sum(
            gbuf[:, 1, :], axis=0
        )

        for rdma in sends:
            rdma.wait_send()

    return pl.pallas_call(
        body,
        out_shape=jax.ShapeDtypeStruct((t,), jnp.float32),
        in_specs=[
            pl.BlockSpec(memory_space=pltpu.VMEM),
            pl.BlockSpec(memory_space=pltpu.VMEM),
            pl.BlockSpec(memory_space=pltpu.VMEM),
        ],
        out_specs=pl.BlockSpec(memory_space=pltpu.VMEM),
        scratch_shapes=[
            pltpu.VMEM((Z, 2, t), jnp.float32),
            pltpu.SemaphoreType.DMA((Z - 1,)),
            pltpu.SemaphoreType.DMA((Z - 1,)),
        ],
        compiler_params=pltpu.CompilerParams(collective_id=0),
    )(x, W, labels)
